# Initial kernel scaffold; baseline (speedup 1.0000x reference)
import functools

import jax
import jax.numpy as jnp
from jax import lax
from jax.experimental import pallas as pl
from jax.experimental.pallas import tpu as pltpu
from jax.experimental.pallas import tpu_sc as plsc

F32 = jnp.float32
I32 = jnp.int32

N0, N1 = 256, 10000
E0, E1 = 8192, 320000
H = 128
ENC = 64
OUT = 3
AE1 = 256
EPS = 1e-5

NC, NS = 2, 16
NW = NC * NS
EPW = E1 // NW
CH = 400
NCHUNK = EPW // CH

BLK_E = 3200
BLK_N = 1000


def _elu(v):
    return jnp.where(v > 0, v, jnp.exp(v) - 1.0)


def _ln(t, g, b):
    mu = jnp.mean(t, axis=-1, keepdims=True)
    var = jnp.mean((t - mu) ** 2, axis=-1, keepdims=True)
    return (t - mu) / jnp.sqrt(var + EPS) * g + b


def _ae_body(xf_ref, w_ref, acc_ref):
    @pl.when(pl.program_id(0) == 0)
    def _():
        acc_ref[...] = jnp.zeros_like(acc_ref)

    acc_ref[...] += jnp.dot(xf_ref[...], w_ref[...], preferred_element_type=F32)


_ae_call = pl.pallas_call(
    _ae_body,
    grid=(16,),
    in_specs=[
        pl.BlockSpec((8, 2048), lambda k: (0, k)),
        pl.BlockSpec((2048, AE1), lambda k: (k, 0)),
    ],
    out_specs=pl.BlockSpec((8, AE1), lambda k: (0, 0)),
    out_shape=jax.ShapeDtypeStruct((8, AE1), F32),
)


def _coarse_body(x_ref, srcc_ref, dstc_ref, dstr_ref, ea_ref, s_ref,
                 permc_ref, permr_ref,
                 e0W1_ref, e0b1_ref, e0W2_ref, e0b2_ref, e0g_ref, e0bn_ref,
                 aeb1_ref, aeW2_ref, aeb2_ref, attnx_ref,
                 n0W1_ref, n0b1_ref, n0W2_ref, n0b2_ref, n0g_ref, n0bn_ref,
                 out_ref):
    x = x_ref[...]
    W1 = e0W1_ref[...]
    xa = jnp.dot(x, W1[:H], preferred_element_type=F32)
    xb = jnp.dot(x, W1[H:2 * H], preferred_element_type=F32)
    col = lax.broadcasted_iota(I32, (E0, N0), 1)
    oh_src = (srcc_ref[...] == col).astype(F32)
    oh_dst = (dstc_ref[...] == col).astype(F32)
    row = lax.broadcasted_iota(I32, (N0, E0), 0)
    oh_dstT = (dstr_ref[...] == row).astype(F32)
    ea = ea_ref[...]
    h = _elu(jnp.dot(oh_src, xa, preferred_element_type=F32)
             + jnp.dot(oh_dst, xb, preferred_element_type=F32)
             + jnp.dot(ea, W1[2 * H:], preferred_element_type=F32)
             + e0b1_ref[...])
    h = jnp.dot(h, e0W2_ref[...], preferred_element_type=F32) + e0b2_ref[...]
    ea0 = _ln(ea + h, e0g_ref[...], e0bn_ref[...])
    sums = jnp.dot(oh_dstT, ea0, preferred_element_type=F32)
    cnt = jnp.maximum(jnp.sum(oh_dstT, axis=-1, keepdims=True), 1.0)
    agg0 = sums / cnt
    x_ae = _elu(s_ref[0:1, :] + aeb1_ref[...])
    g_ae = jnp.dot(x_ae, aeW2_ref[...], preferred_element_type=F32) + aeb2_ref[...]
    logit = jnp.dot(x, attnx_ref[...], preferred_element_type=F32)
    e = jnp.exp(logit - jnp.max(logit))
    score = e / jnp.sum(e)
    n0W1 = n0W1_ref[...]
    pre = (jnp.dot(x, n0W1[:H], preferred_element_type=F32)
           + jnp.dot(agg0, n0W1[H:2 * H], preferred_element_type=F32)
           + score * jnp.dot(g_ae, n0W1[2 * H:], preferred_element_type=F32)
           + n0b1_ref[...])
    h = _elu(pre)
    h = jnp.dot(h, n0W2_ref[...], preferred_element_type=F32) + n0b2_ref[...]
    x0 = _ln(x + h, n0g_ref[...], n0bn_ref[...])
    pc = permc_ref[...]
    pr = permr_ref[...]
    jj = lax.broadcasted_iota(I32, (N0, N0), 1)
    maxidx = jnp.max(jnp.where(pc == pr, jj, -1), axis=-1, keepdims=True)
    win = (lax.broadcasted_iota(I32, (N0, 1), 0) == maxidx).astype(F32)
    out_ref[...] = x0 * win


_coarse_shapes = [
    (N0, H), (E0, 1), (E0, 1), (1, E0), (E0, H), (8, AE1),
    (N0, 1), (1, N0),
    (3 * H, H), (1, H), (H, H), (1, H), (1, H), (1, H),
    (1, AE1), (AE1, ENC), (1, ENC), (H, 1),
    (2 * H + ENC, H), (1, H), (H, H), (1, H), (1, H), (1, H),
]

_coarse_call = pl.pallas_call(
    _coarse_body,
    in_specs=[pl.BlockSpec(s, lambda: (0,) * len(s)) for s in _coarse_shapes],
    out_specs=pl.BlockSpec((N0, H), lambda: (0, 0)),
    out_shape=jax.ShapeDtypeStruct((N0, H), F32),
)


def _unpool_body(x0_ref, permr_ref, skip_ref, W1_ref, xf_ref, xs1_ref, xs2_ref):
    b = pl.program_id(0)
    rows = lax.broadcasted_iota(I32, (BLK_N, N0), 0) + b * BLK_N
    oh = (rows == permr_ref[...]).astype(F32)
    xf = jnp.dot(oh, x0_ref[...], preferred_element_type=F32) + skip_ref[...]
    xf_ref[...] = xf
    W1 = W1_ref[...]
    xs1_ref[...] = jnp.dot(xf, W1[:H], preferred_element_type=F32)
    xs2_ref[...] = jnp.dot(xf, W1[H:2 * H], preferred_element_type=F32)


_unpool_call = pl.pallas_call(
    _unpool_body,
    grid=(N1 // BLK_N,),
    in_specs=[
        pl.BlockSpec((N0, H), lambda i: (0, 0)),
        pl.BlockSpec((1, N0), lambda i: (0, 0)),
        pl.BlockSpec((BLK_N, H), lambda i: (i, 0)),
        pl.BlockSpec((3 * H, H), lambda i: (0, 0)),
    ],
    out_specs=[pl.BlockSpec((BLK_N, H), lambda i: (i, 0))] * 3,
    out_shape=[jax.ShapeDtypeStruct((N1, H), F32)] * 3,
)


_sc_mesh = plsc.VectorSubcoreMesh(
    core_axis_name="c", subcore_axis_name="s", num_cores=NC, num_subcores=NS)


@functools.partial(
    pl.kernel,
    out_type=(jax.ShapeDtypeStruct((E1, H), F32),
              jax.ShapeDtypeStruct((E1, H), F32)),
    mesh=_sc_mesh,
    scratch_types=[
        pltpu.VMEM((CH,), I32),
        pltpu.VMEM((CH,), I32),
        pltpu.VMEM((CH, H), F32),
        pltpu.VMEM((CH, H), F32),
        pltpu.SemaphoreType.DMA,
        pltpu.SemaphoreType.DMA,
    ],
)
def _sc_gather(xs1_hbm, xs2_hbm, src_hbm, dst_hbm, g1_hbm, g2_hbm,
               sidx, didx, buf1, buf2, sem1, sem2):
    wid = lax.axis_index("s") * NC + lax.axis_index("c")
    base = wid * EPW

    def body(j, carry):
        off = base + j * CH
        pltpu.sync_copy(src_hbm.at[pl.ds(off, CH)], sidx)
        pltpu.sync_copy(dst_hbm.at[pl.ds(off, CH)], didx)
        cp1 = pltpu.async_copy(xs1_hbm.at[sidx], buf1, sem1)
        cp2 = pltpu.async_copy(xs2_hbm.at[didx], buf2, sem2)
        cp1.wait()
        cp2.wait()
        pltpu.sync_copy(buf1, g1_hbm.at[pl.ds(off, CH)])
        pltpu.sync_copy(buf2, g2_hbm.at[pl.ds(off, CH)])
        return carry

    lax.fori_loop(0, NCHUNK, body, 0)


def _edge_body(g1_ref, g2_ref, ea_ref, W1_ref, b1_ref, W2_ref, b2_ref,
               g_ref, bn_ref, out_ref):
    ea = ea_ref[...]
    h = _elu(g1_ref[...] + g2_ref[...]
             + jnp.dot(ea, W1_ref[...][2 * H:], preferred_element_type=F32)
             + b1_ref[...])
    h = jnp.dot(h, W2_ref[...], preferred_element_type=F32) + b2_ref[...]
    out_ref[...] = _ln(ea + h, g_ref[...], bn_ref[...])


_edge_call = pl.pallas_call(
    _edge_body,
    grid=(E1 // BLK_E,),
    in_specs=[
        pl.BlockSpec((BLK_E, H), lambda i: (i, 0)),
        pl.BlockSpec((BLK_E, H), lambda i: (i, 0)),
        pl.BlockSpec((BLK_E, H), lambda i: (i, 0)),
        pl.BlockSpec((3 * H, H), lambda i: (0, 0)),
        pl.BlockSpec((1, H), lambda i: (0, 0)),
        pl.BlockSpec((H, H), lambda i: (0, 0)),
        pl.BlockSpec((1, H), lambda i: (0, 0)),
        pl.BlockSpec((1, H), lambda i: (0, 0)),
        pl.BlockSpec((1, H), lambda i: (0, 0)),
    ],
    out_specs=pl.BlockSpec((BLK_E, H), lambda i: (i, 0)),
    out_shape=jax.ShapeDtypeStruct((E1, H), F32),
)


@functools.partial(
    pl.kernel,
    out_type=(jax.ShapeDtypeStruct((NC, N1, H), F32),
              jax.ShapeDtypeStruct((NC, N1, 16), F32)),
    mesh=_sc_mesh,
    scratch_types=[
        pltpu.VMEM((CH,), I32),
        pltpu.VMEM((CH, H), F32),
        pltpu.VMEM((CH, 16), F32),
        pltpu.VMEM_SHARED((N1, H), F32),
        pltpu.VMEM_SHARED((N1, 16), F32),
    ],
)
def _sc_scatter(ea1_hbm, dst_hbm, ones_hbm, zacc_hbm, zcnt_hbm,
                acc_out, cnt_out, idxb, buf, onesb, acc_sh, cnt_sh):
    c = lax.axis_index("c")
    s = lax.axis_index("s")

    @pl.when(s == 0)
    def _():
        pltpu.sync_copy(zacc_hbm, acc_sh)
        pltpu.sync_copy(zcnt_hbm, cnt_sh)

    pltpu.sync_copy(ones_hbm, onesb)
    plsc.subcore_barrier()
    base = (c * NS + s) * EPW

    def body(j, carry):
        off = base + j * CH
        pltpu.sync_copy(dst_hbm.at[pl.ds(off, CH)], idxb)
        pltpu.sync_copy(ea1_hbm.at[pl.ds(off, CH)], buf)
        pltpu.sync_copy(buf, acc_sh.at[idxb], add=True)
        pltpu.sync_copy(onesb, cnt_sh.at[idxb], add=True)
        return carry

    lax.fori_loop(0, NCHUNK, body, 0)
    plsc.subcore_barrier()
    rows = N1 // NS
    r0 = s * rows
    pltpu.sync_copy(acc_sh.at[pl.ds(r0, rows)], acc_out.at[c, pl.ds(r0, rows)])
    pltpu.sync_copy(cnt_sh.at[pl.ds(r0, rows)], cnt_out.at[c, pl.ds(r0, rows)])


def _decode_body(xf_ref, acc_ref, cnt_ref, W1_ref, b1_ref, W2_ref, b2_ref,
                 g_ref, bn_ref, dW1_ref, db1_ref, dW2_ref, db2_ref, out_ref):
    xf = xf_ref[...]
    ssum = acc_ref[0] + acc_ref[1]
    cnt = jnp.maximum(cnt_ref[0, :, 0:1] + cnt_ref[1, :, 0:1], 1.0)
    agg = ssum / cnt
    W1 = W1_ref[...]
    h = _elu(jnp.dot(xf, W1[:H], preferred_element_type=F32)
             + jnp.dot(agg, W1[H:], preferred_element_type=F32)
             + b1_ref[...])
    h = jnp.dot(h, W2_ref[...], preferred_element_type=F32) + b2_ref[...]
    xf2 = _ln(xf + h, g_ref[...], bn_ref[...])
    d = _elu(jnp.dot(xf2, dW1_ref[...], preferred_element_type=F32) + db1_ref[...])
    out_ref[...] = jnp.dot(d, dW2_ref[...], preferred_element_type=F32) + db2_ref[...]


_decode_call = pl.pallas_call(
    _decode_body,
    grid=(N1 // BLK_N,),
    in_specs=[
        pl.BlockSpec((BLK_N, H), lambda i: (i, 0)),
        pl.BlockSpec((NC, BLK_N, H), lambda i: (0, i, 0)),
        pl.BlockSpec((NC, BLK_N, 16), lambda i: (0, i, 0)),
        pl.BlockSpec((2 * H, H), lambda i: (0, 0)),
        pl.BlockSpec((1, H), lambda i: (0, 0)),
        pl.BlockSpec((H, H), lambda i: (0, 0)),
        pl.BlockSpec((1, H), lambda i: (0, 0)),
        pl.BlockSpec((1, H), lambda i: (0, 0)),
        pl.BlockSpec((1, H), lambda i: (0, 0)),
        pl.BlockSpec((H, H), lambda i: (0, 0)),
        pl.BlockSpec((1, H), lambda i: (0, 0)),
        pl.BlockSpec((H, H), lambda i: (0, 0)),
        pl.BlockSpec((1, H), lambda i: (0, 0)),
    ],
    out_specs=pl.BlockSpec((BLK_N, H), lambda i: (i, 0)),
    out_shape=jax.ShapeDtypeStruct((N1, H), F32),
)


def _sc_gather_fn(xs1, xs2, src, dst):
    return _sc_gather(xs1, xs2, src, dst)


def _sc_scatter_fn(ea1, dst, ones, zacc, zcnt):
    return _sc_scatter(ea1, dst, ones, zacc, zcnt)


def kernel(x, edge_index, edge_attr, x_fine_skip, edge_index_fine,
           edge_attr_fine, perm, params):
    p = params
    src0 = edge_index[0].astype(I32)
    dst0 = edge_index[1].astype(I32)
    srcf = edge_index_fine[0].astype(I32)
    dstf = edge_index_fine[1].astype(I32)
    perm = perm.astype(I32)

    x_flat8 = jnp.concatenate(
        [x.reshape(1, N0 * H), jnp.zeros((7, N0 * H), F32)], axis=0)
    s_ae = _ae_call(x_flat8, p['ae_W1'])

    x0f = _coarse_call(
        x, src0.reshape(E0, 1), dst0.reshape(E0, 1), dst0.reshape(1, E0),
        edge_attr, s_ae, perm.reshape(N0, 1), perm.reshape(1, N0),
        p['e0_W1'], p['e0_b1'].reshape(1, H), p['e0_W2'],
        p['e0_b2'].reshape(1, H), p['e0_g'].reshape(1, H),
        p['e0_bn'].reshape(1, H),
        p['ae_b1'].reshape(1, AE1), p['ae_W2'], p['ae_b2'].reshape(1, ENC),
        p['attn_w'][:H].reshape(H, 1),
        p['n0_W1'], p['n0_b1'].reshape(1, H), p['n0_W2'],
        p['n0_b2'].reshape(1, H), p['n0_g'].reshape(1, H),
        p['n0_bn'].reshape(1, H))

    xf, xs1, xs2 = _unpool_call(x0f, perm.reshape(1, N0), x_fine_skip,
                                p['e1_W1'])

    g1, g2 = _sc_gather_fn(xs1, xs2, srcf, dstf)

    ea1 = _edge_call(g1, g2, edge_attr_fine, p['e1_W1'],
                     p['e1_b1'].reshape(1, H), p['e1_W2'],
                     p['e1_b2'].reshape(1, H), p['e1_g'].reshape(1, H),
                     p['e1_bn'].reshape(1, H))

    acc, cnt = _sc_scatter_fn(
        ea1, dstf, jnp.ones((CH, 16), F32),
        jnp.zeros((N1, H), F32), jnp.zeros((N1, 16), F32))

    dW2p = jnp.zeros((H, H), F32).at[:, :OUT].set(p['dec_W2'])
    db2p = jnp.zeros((1, H), F32).at[0, :OUT].set(p['dec_b2'])
    out_full = _decode_call(
        xf, acc, cnt, p['n1_W1'], p['n1_b1'].reshape(1, H), p['n1_W2'],
        p['n1_b2'].reshape(1, H), p['n1_g'].reshape(1, H),
        p['n1_bn'].reshape(1, H), p['dec_W1'], p['dec_b1'].reshape(1, H),
        dW2p, db2p)
    return out_full[:, :OUT]


# speedup vs baseline: 2.2667x; 2.2667x over previous
import functools

import jax
import jax.numpy as jnp
from jax import lax
from jax.experimental import pallas as pl
from jax.experimental.pallas import tpu as pltpu
from jax.experimental.pallas import tpu_sc as plsc

F32 = jnp.float32
I32 = jnp.int32

N0, N1 = 256, 10000
E0, E1 = 8192, 320000
H = 128
ENC = 64
OUT = 3
AE1 = 256
EPS = 1e-5

NC, NS = 2, 16
NW = NC * NS
EPW = E1 // NW
CH = 80
NCHUNK = EPW // CH

BLK_E = 3200
BLK_N = 1000


def _elu(v):
    return jnp.where(v > 0, v, jnp.exp(v) - 1.0)


def _ln(t, g, b):
    mu = jnp.mean(t, axis=-1, keepdims=True)
    var = jnp.mean((t - mu) ** 2, axis=-1, keepdims=True)
    return (t - mu) / jnp.sqrt(var + EPS) * g + b


def _ae_body(xf_ref, w_ref, acc_ref):
    @pl.when(pl.program_id(0) == 0)
    def _():
        acc_ref[...] = jnp.zeros_like(acc_ref)

    acc_ref[...] += jnp.dot(xf_ref[...], w_ref[...], preferred_element_type=F32)


_ae_call = pl.pallas_call(
    _ae_body,
    grid=(16,),
    in_specs=[
        pl.BlockSpec((8, 2048), lambda k: (0, k)),
        pl.BlockSpec((2048, AE1), lambda k: (k, 0)),
    ],
    out_specs=pl.BlockSpec((8, AE1), lambda k: (0, 0)),
    out_shape=jax.ShapeDtypeStruct((8, AE1), F32),
)


def _coarse_body(x_ref, srcc_ref, dstc_ref, dstr_ref, ea_ref, s_ref,
                 permc_ref, permr_ref,
                 e0W1_ref, e0b1_ref, e0W2_ref, e0b2_ref, e0g_ref, e0bn_ref,
                 aeb1_ref, aeW2_ref, aeb2_ref, attnx_ref,
                 n0W1_ref, n0b1_ref, n0W2_ref, n0b2_ref, n0g_ref, n0bn_ref,
                 out_ref):
    x = x_ref[...]
    W1 = e0W1_ref[...]
    xa = jnp.dot(x, W1[:H], preferred_element_type=F32)
    xb = jnp.dot(x, W1[H:2 * H], preferred_element_type=F32)
    col = lax.broadcasted_iota(I32, (E0, N0), 1)
    oh_src = (srcc_ref[...] == col).astype(F32)
    oh_dst = (dstc_ref[...] == col).astype(F32)
    row = lax.broadcasted_iota(I32, (N0, E0), 0)
    oh_dstT = (dstr_ref[...] == row).astype(F32)
    ea = ea_ref[...]
    h = _elu(jnp.dot(oh_src, xa, preferred_element_type=F32)
             + jnp.dot(oh_dst, xb, preferred_element_type=F32)
             + jnp.dot(ea, W1[2 * H:], preferred_element_type=F32)
             + e0b1_ref[...])
    h = jnp.dot(h, e0W2_ref[...], preferred_element_type=F32) + e0b2_ref[...]
    ea0 = _ln(ea + h, e0g_ref[...], e0bn_ref[...])
    sums = jnp.dot(oh_dstT, ea0, preferred_element_type=F32)
    cnt = jnp.maximum(jnp.sum(oh_dstT, axis=-1, keepdims=True), 1.0)
    agg0 = sums / cnt
    x_ae = _elu(s_ref[0:1, :] + aeb1_ref[...])
    g_ae = jnp.dot(x_ae, aeW2_ref[...], preferred_element_type=F32) + aeb2_ref[...]
    logit = jnp.dot(x, attnx_ref[...], preferred_element_type=F32)
    e = jnp.exp(logit - jnp.max(logit))
    score = e / jnp.sum(e)
    n0W1 = n0W1_ref[...]
    pre = (jnp.dot(x, n0W1[:H], preferred_element_type=F32)
           + jnp.dot(agg0, n0W1[H:2 * H], preferred_element_type=F32)
           + score * jnp.dot(g_ae, n0W1[2 * H:], preferred_element_type=F32)
           + n0b1_ref[...])
    h = _elu(pre)
    h = jnp.dot(h, n0W2_ref[...], preferred_element_type=F32) + n0b2_ref[...]
    x0 = _ln(x + h, n0g_ref[...], n0bn_ref[...])
    pc = permc_ref[...]
    pr = permr_ref[...]
    jj = lax.broadcasted_iota(I32, (N0, N0), 1)
    winidx = jnp.max(jnp.where(pc == pr, jj, -1), axis=-1, keepdims=True)
    win = (lax.broadcasted_iota(I32, (N0, 1), 0) == winidx).astype(F32)
    out_ref[...] = x0 * win


_coarse_shapes = [
    (N0, H), (E0, 1), (E0, 1), (1, E0), (E0, H), (8, AE1),
    (N0, 1), (1, N0),
    (3 * H, H), (1, H), (H, H), (1, H), (1, H), (1, H),
    (1, AE1), (AE1, ENC), (1, ENC), (H, 1),
    (2 * H + ENC, H), (1, H), (H, H), (1, H), (1, H), (1, H),
]

_coarse_call = pl.pallas_call(
    _coarse_body,
    in_specs=[pl.BlockSpec(s, lambda: (0,) * len(s)) for s in _coarse_shapes],
    out_specs=pl.BlockSpec((N0, H), lambda: (0, 0)),
    out_shape=jax.ShapeDtypeStruct((N0, H), F32),
)


def _unpool_body(x0_ref, permr_ref, skip_ref, W1_ref, xf_ref, xs1_ref, xs2_ref):
    b = pl.program_id(0)
    rows = lax.broadcasted_iota(I32, (BLK_N, N0), 0) + b * BLK_N
    oh = (rows == permr_ref[...]).astype(F32)
    xf = jnp.dot(oh, x0_ref[...], preferred_element_type=F32) + skip_ref[...]
    xf_ref[...] = xf
    W1 = W1_ref[...]
    xs1_ref[...] = jnp.dot(xf, W1[:H], preferred_element_type=F32)
    xs2_ref[...] = jnp.dot(xf, W1[H:2 * H], preferred_element_type=F32)


_unpool_call = pl.pallas_call(
    _unpool_body,
    grid=(N1 // BLK_N,),
    in_specs=[
        pl.BlockSpec((N0, H), lambda i: (0, 0)),
        pl.BlockSpec((1, N0), lambda i: (0, 0)),
        pl.BlockSpec((BLK_N, H), lambda i: (i, 0)),
        pl.BlockSpec((3 * H, H), lambda i: (0, 0)),
    ],
    out_specs=[pl.BlockSpec((BLK_N, H), lambda i: (i, 0))] * 3,
    out_shape=[jax.ShapeDtypeStruct((N1, H), F32)] * 3,
)


@functools.cache
def _make_sc_gather():
    mesh = plsc.VectorSubcoreMesh(
        core_axis_name="c", subcore_axis_name="s",
        num_cores=NC, num_subcores=NS)

    @functools.partial(
        pl.kernel,
        out_type=(jax.ShapeDtypeStruct((E1, H), F32),
                  jax.ShapeDtypeStruct((E1, H), F32)),
        mesh=mesh,
        scratch_types=[
            pltpu.VMEM((CH,), I32),
            pltpu.VMEM((CH,), I32),
            pltpu.VMEM((CH, H), F32),
            pltpu.VMEM((CH, H), F32),
            pltpu.SemaphoreType.DMA,
            pltpu.SemaphoreType.DMA,
        ],
    )
    def sc_gather(xs1_hbm, xs2_hbm, src_hbm, dst_hbm, g1_hbm, g2_hbm,
                  sidx, didx, buf1, buf2, sem1, sem2):
        wid = lax.axis_index("s") * NC + lax.axis_index("c")
        base = wid * EPW

        def body(j, carry):
            off = base + j * CH
            pltpu.sync_copy(src_hbm.at[pl.ds(off, CH)], sidx)
            pltpu.sync_copy(dst_hbm.at[pl.ds(off, CH)], didx)
            cp1 = pltpu.async_copy(xs1_hbm.at[sidx], buf1, sem1)
            cp2 = pltpu.async_copy(xs2_hbm.at[didx], buf2, sem2)
            cp1.wait()
            cp2.wait()
            pltpu.sync_copy(buf1, g1_hbm.at[pl.ds(off, CH)])
            pltpu.sync_copy(buf2, g2_hbm.at[pl.ds(off, CH)])
            return carry

        lax.fori_loop(0, NCHUNK, body, 0)

    return sc_gather


def _edge_body(g1_ref, g2_ref, ea_ref, W1_ref, b1_ref, W2_ref, b2_ref,
               g_ref, bn_ref, out_ref):
    ea = ea_ref[...]
    h = _elu(g1_ref[...] + g2_ref[...]
             + jnp.dot(ea, W1_ref[...][2 * H:], preferred_element_type=F32)
             + b1_ref[...])
    h = jnp.dot(h, W2_ref[...], preferred_element_type=F32) + b2_ref[...]
    out_ref[...] = _ln(ea + h, g_ref[...], bn_ref[...])


_edge_call = pl.pallas_call(
    _edge_body,
    grid=(E1 // BLK_E,),
    in_specs=[
        pl.BlockSpec((BLK_E, H), lambda i: (i, 0)),
        pl.BlockSpec((BLK_E, H), lambda i: (i, 0)),
        pl.BlockSpec((BLK_E, H), lambda i: (i, 0)),
        pl.BlockSpec((3 * H, H), lambda i: (0, 0)),
        pl.BlockSpec((1, H), lambda i: (0, 0)),
        pl.BlockSpec((H, H), lambda i: (0, 0)),
        pl.BlockSpec((1, H), lambda i: (0, 0)),
        pl.BlockSpec((1, H), lambda i: (0, 0)),
        pl.BlockSpec((1, H), lambda i: (0, 0)),
    ],
    out_specs=pl.BlockSpec((BLK_E, H), lambda i: (i, 0)),
    out_shape=jax.ShapeDtypeStruct((E1, H), F32),
)


EPW1 = E1 // NS
CHS = 80
NCHUNKS = EPW1 // CHS
RPT = (N1 // NS) // 8 * 8
CPR = 48
NTAIL = N1 - RPT * NS


@functools.cache
def _make_sc_scatter():
    mesh = plsc.VectorSubcoreMesh(
        core_axis_name="c", subcore_axis_name="s",
        num_cores=1, num_subcores=NS)

    @functools.partial(
        pl.kernel,
        out_type=(jax.ShapeDtypeStruct((N1, H), F32),
                  jax.ShapeDtypeStruct((N1, H), F32)),
        mesh=mesh,
        scratch_types=[
            pltpu.VMEM((CHS,), I32),
            pltpu.VMEM((CHS, H), F32),
            pltpu.VMEM((CHS, H), F32),
            pltpu.VMEM((CPR,), I32),
            pltpu.VMEM((NTAIL,), I32),
            pltpu.VMEM((CPR, H), F32),
            pltpu.VMEM_SHARED((N1, H), F32),
        ],
    )
    def sc_scatter(ea1_hbm, dst_hbm, iota_hbm, zacc_hbm, ones_hbm,
                   acc_out, cnt_out, idxb, buf, onesb, ridx, ridx2, abuf,
                   acc_sh):
        s = lax.axis_index("s")
        r0 = s * RPT
        base = s * EPW1

        def zero_fill():
            for k in range(RPT // CPR):
                pltpu.sync_copy(iota_hbm.at[pl.ds(r0 + k * CPR, CPR)], ridx)
                pltpu.sync_copy(abuf, acc_sh.at[ridx])

            @pl.when(s == NS - 1)
            def _():
                t0 = RPT * NS
                pltpu.sync_copy(iota_hbm.at[pl.ds(t0, NTAIL)], ridx2)
                pltpu.sync_copy(abuf.at[pl.ds(0, NTAIL)], acc_sh.at[ridx2])

        def drain(out_hbm):
            for k in range(RPT // CPR):
                pltpu.sync_copy(iota_hbm.at[pl.ds(r0 + k * CPR, CPR)], ridx)
                pltpu.sync_copy(acc_sh.at[ridx], abuf)
                pltpu.sync_copy(abuf, out_hbm.at[pl.ds(r0 + k * CPR, CPR)])

            @pl.when(s == NS - 1)
            def _():
                t0 = RPT * NS
                pltpu.sync_copy(iota_hbm.at[pl.ds(t0, NTAIL)], ridx2)
                pltpu.sync_copy(acc_sh.at[ridx2], abuf.at[pl.ds(0, NTAIL)])
                pltpu.sync_copy(abuf.at[pl.ds(0, NTAIL)],
                                out_hbm.at[pl.ds(t0, NTAIL)])

        pltpu.sync_copy(zacc_hbm, abuf)
        zero_fill()
        plsc.subcore_barrier()

        def body(j, carry):
            off = base + j * CHS
            pltpu.sync_copy(dst_hbm.at[pl.ds(off, CHS)], idxb)
            pltpu.sync_copy(ea1_hbm.at[pl.ds(off, CHS)], buf)
            pltpu.sync_copy(buf, acc_sh.at[idxb], add=True)
            return carry

        lax.fori_loop(0, NCHUNKS, body, 0)
        plsc.subcore_barrier()
        drain(acc_out)

        plsc.subcore_barrier()
        pltpu.sync_copy(zacc_hbm, abuf)
        zero_fill()
        pltpu.sync_copy(ones_hbm, onesb)
        plsc.subcore_barrier()

        def body2(j, carry):
            off = base + j * CHS
            pltpu.sync_copy(dst_hbm.at[pl.ds(off, CHS)], idxb)
            pltpu.sync_copy(onesb, acc_sh.at[idxb], add=True)
            return carry

        lax.fori_loop(0, NCHUNKS, body2, 0)
        plsc.subcore_barrier()
        drain(cnt_out)

    return sc_scatter


def _decode_body(xf_ref, acc_ref, cnt_ref, W1_ref, b1_ref, W2_ref,
                 b2_ref, g_ref, bn_ref, dW1_ref, db1_ref, dW2_ref, db2_ref,
                 out_ref):
    xf = xf_ref[...]
    cnt = jnp.maximum(cnt_ref[...][:, 0:1], 1.0)
    agg = acc_ref[...] / cnt
    W1 = W1_ref[...]
    h = _elu(jnp.dot(xf, W1[:H], preferred_element_type=F32)
             + jnp.dot(agg, W1[H:], preferred_element_type=F32)
             + b1_ref[...])
    h = jnp.dot(h, W2_ref[...], preferred_element_type=F32) + b2_ref[...]
    xf2 = _ln(xf + h, g_ref[...], bn_ref[...])
    d = _elu(jnp.dot(xf2, dW1_ref[...], preferred_element_type=F32) + db1_ref[...])
    out_ref[...] = jnp.dot(d, dW2_ref[...], preferred_element_type=F32) + db2_ref[...]


_decode_call = pl.pallas_call(
    _decode_body,
    grid=(N1 // BLK_N,),
    in_specs=[
        pl.BlockSpec((BLK_N, H), lambda i: (i, 0)),
        pl.BlockSpec((BLK_N, H), lambda i: (i, 0)),
        pl.BlockSpec((BLK_N, H), lambda i: (i, 0)),
        pl.BlockSpec((2 * H, H), lambda i: (0, 0)),
        pl.BlockSpec((1, H), lambda i: (0, 0)),
        pl.BlockSpec((H, H), lambda i: (0, 0)),
        pl.BlockSpec((1, H), lambda i: (0, 0)),
        pl.BlockSpec((1, H), lambda i: (0, 0)),
        pl.BlockSpec((1, H), lambda i: (0, 0)),
        pl.BlockSpec((H, H), lambda i: (0, 0)),
        pl.BlockSpec((1, H), lambda i: (0, 0)),
        pl.BlockSpec((H, H), lambda i: (0, 0)),
        pl.BlockSpec((1, H), lambda i: (0, 0)),
    ],
    out_specs=pl.BlockSpec((BLK_N, H), lambda i: (i, 0)),
    out_shape=jax.ShapeDtypeStruct((N1, H), F32),
)


def _sc_gather_fn(xs1, xs2, src, dst):
    return _make_sc_gather()(xs1, xs2, src, dst)


def _sc_scatter_fn(ea1, dst, iota, zacc, ones):
    return _make_sc_scatter()(ea1, dst, iota, zacc, ones)


def kernel(x, edge_index, edge_attr, x_fine_skip, edge_index_fine,
           edge_attr_fine, perm, params):
    p = params
    src0 = edge_index[0].astype(I32)
    dst0 = edge_index[1].astype(I32)
    srcf = edge_index_fine[0].astype(I32)
    dstf = edge_index_fine[1].astype(I32)
    perm = perm.astype(I32)

    x_flat8 = jnp.concatenate(
        [x.reshape(1, N0 * H), jnp.zeros((7, N0 * H), F32)], axis=0)
    s_ae = _ae_call(x_flat8, p['ae_W1'])

    x0f = _coarse_call(
        x, src0.reshape(E0, 1), dst0.reshape(E0, 1), dst0.reshape(1, E0),
        edge_attr, s_ae, perm.reshape(N0, 1), perm.reshape(1, N0),
        p['e0_W1'], p['e0_b1'].reshape(1, H), p['e0_W2'],
        p['e0_b2'].reshape(1, H), p['e0_g'].reshape(1, H),
        p['e0_bn'].reshape(1, H),
        p['ae_b1'].reshape(1, AE1), p['ae_W2'], p['ae_b2'].reshape(1, ENC),
        p['attn_w'][:H].reshape(H, 1),
        p['n0_W1'], p['n0_b1'].reshape(1, H), p['n0_W2'],
        p['n0_b2'].reshape(1, H), p['n0_g'].reshape(1, H),
        p['n0_bn'].reshape(1, H))

    xf, xs1, xs2 = _unpool_call(x0f, perm.reshape(1, N0), x_fine_skip,
                                p['e1_W1'])

    g1, g2 = _sc_gather_fn(xs1, xs2, srcf, dstf)

    ea1 = _edge_call(g1, g2, edge_attr_fine, p['e1_W1'],
                     p['e1_b1'].reshape(1, H), p['e1_W2'],
                     p['e1_b2'].reshape(1, H), p['e1_g'].reshape(1, H),
                     p['e1_bn'].reshape(1, H))

    acc, cnt = _sc_scatter_fn(ea1, dstf, jnp.arange(N1, dtype=I32),
                              jnp.zeros((CPR, H), F32),
                              jnp.ones((CHS, H), F32))

    dW2p = jnp.zeros((H, H), F32).at[:, :OUT].set(p['dec_W2'])
    db2p = jnp.zeros((1, H), F32).at[0, :OUT].set(p['dec_b2'])
    out_full = _decode_call(
        xf, acc, cnt,
        p['n1_W1'], p['n1_b1'].reshape(1, H), p['n1_W2'],
        p['n1_b2'].reshape(1, H), p['n1_g'].reshape(1, H),
        p['n1_bn'].reshape(1, H), p['dec_W1'], p['dec_b1'].reshape(1, H),
        dW2p, db2p)
    return out_full[:, :OUT]


# speedup vs baseline: 2.9117x; 1.2846x over previous
import functools

import jax
import jax.numpy as jnp
from jax import lax
from jax.experimental import pallas as pl
from jax.experimental.pallas import tpu as pltpu
from jax.experimental.pallas import tpu_sc as plsc

F32 = jnp.float32
I32 = jnp.int32

N0, N1 = 256, 10000
E0, E1 = 8192, 320000
H = 128
ENC = 64
OUT = 3
AE1 = 256
EPS = 1e-5

NC, NS = 2, 16
NW = NC * NS
EPW = E1 // NW
CH = 80
NCHUNK = EPW // CH

BLK_E = 3200
BLK_N = 1000


def _elu(v):
    return jnp.where(v > 0, v, jnp.exp(v) - 1.0)


def _ln(t, g, b):
    mu = jnp.mean(t, axis=-1, keepdims=True)
    var = jnp.mean((t - mu) ** 2, axis=-1, keepdims=True)
    return (t - mu) / jnp.sqrt(var + EPS) * g + b


def _ae_body(xf_ref, w_ref, acc_ref):
    @pl.when(pl.program_id(0) == 0)
    def _():
        acc_ref[...] = jnp.zeros_like(acc_ref)

    acc_ref[...] += jnp.dot(xf_ref[...], w_ref[...], preferred_element_type=F32)


_ae_call = pl.pallas_call(
    _ae_body,
    grid=(16,),
    in_specs=[
        pl.BlockSpec((8, 2048), lambda k: (0, k)),
        pl.BlockSpec((2048, AE1), lambda k: (k, 0)),
    ],
    out_specs=pl.BlockSpec((8, AE1), lambda k: (0, 0)),
    out_shape=jax.ShapeDtypeStruct((8, AE1), F32),
)


def _coarse_body(x_ref, srcc_ref, dstc_ref, dstr_ref, ea_ref, s_ref,
                 permc_ref, permr_ref,
                 e0W1_ref, e0b1_ref, e0W2_ref, e0b2_ref, e0g_ref, e0bn_ref,
                 aeb1_ref, aeW2_ref, aeb2_ref, attnx_ref,
                 n0W1_ref, n0b1_ref, n0W2_ref, n0b2_ref, n0g_ref, n0bn_ref,
                 out_ref):
    x = x_ref[...]
    W1 = e0W1_ref[...]
    xa = jnp.dot(x, W1[:H], preferred_element_type=F32)
    xb = jnp.dot(x, W1[H:2 * H], preferred_element_type=F32)
    col = lax.broadcasted_iota(I32, (E0, N0), 1)
    oh_src = (srcc_ref[...] == col).astype(F32)
    oh_dst = (dstc_ref[...] == col).astype(F32)
    row = lax.broadcasted_iota(I32, (N0, E0), 0)
    oh_dstT = (dstr_ref[...] == row).astype(F32)
    ea = ea_ref[...]
    h = _elu(jnp.dot(oh_src, xa, preferred_element_type=F32)
             + jnp.dot(oh_dst, xb, preferred_element_type=F32)
             + jnp.dot(ea, W1[2 * H:], preferred_element_type=F32)
             + e0b1_ref[...])
    h = jnp.dot(h, e0W2_ref[...], preferred_element_type=F32) + e0b2_ref[...]
    ea0 = _ln(ea + h, e0g_ref[...], e0bn_ref[...])
    sums = jnp.dot(oh_dstT, ea0, preferred_element_type=F32)
    cnt = jnp.maximum(jnp.sum(oh_dstT, axis=-1, keepdims=True), 1.0)
    agg0 = sums / cnt
    x_ae = _elu(s_ref[0:1, :] + aeb1_ref[...])
    g_ae = jnp.dot(x_ae, aeW2_ref[...], preferred_element_type=F32) + aeb2_ref[...]
    logit = jnp.dot(x, attnx_ref[...], preferred_element_type=F32)
    e = jnp.exp(logit - jnp.max(logit))
    score = e / jnp.sum(e)
    n0W1 = n0W1_ref[...]
    pre = (jnp.dot(x, n0W1[:H], preferred_element_type=F32)
           + jnp.dot(agg0, n0W1[H:2 * H], preferred_element_type=F32)
           + score * jnp.dot(g_ae, n0W1[2 * H:], preferred_element_type=F32)
           + n0b1_ref[...])
    h = _elu(pre)
    h = jnp.dot(h, n0W2_ref[...], preferred_element_type=F32) + n0b2_ref[...]
    x0 = _ln(x + h, n0g_ref[...], n0bn_ref[...])
    pc = permc_ref[...]
    pr = permr_ref[...]
    jj = lax.broadcasted_iota(I32, (N0, N0), 1)
    winidx = jnp.max(jnp.where(pc == pr, jj, -1), axis=-1, keepdims=True)
    win = (lax.broadcasted_iota(I32, (N0, 1), 0) == winidx).astype(F32)
    out_ref[...] = x0 * win


_coarse_shapes = [
    (N0, H), (E0, 1), (E0, 1), (1, E0), (E0, H), (8, AE1),
    (N0, 1), (1, N0),
    (3 * H, H), (1, H), (H, H), (1, H), (1, H), (1, H),
    (1, AE1), (AE1, ENC), (1, ENC), (H, 1),
    (2 * H + ENC, H), (1, H), (H, H), (1, H), (1, H), (1, H),
]

_coarse_call = pl.pallas_call(
    _coarse_body,
    in_specs=[pl.BlockSpec(s, lambda: (0,) * len(s)) for s in _coarse_shapes],
    out_specs=pl.BlockSpec((N0, H), lambda: (0, 0)),
    out_shape=jax.ShapeDtypeStruct((N0, H), F32),
)


def _unpool_body(x0_ref, permr_ref, skip_ref, W1_ref, xf_ref, xs1_ref, xs2_ref):
    b = pl.program_id(0)
    rows = lax.broadcasted_iota(I32, (BLK_N, N0), 0) + b * BLK_N
    oh = (rows == permr_ref[...]).astype(F32)
    xf = jnp.dot(oh, x0_ref[...], preferred_element_type=F32) + skip_ref[...]
    xf_ref[...] = xf
    W1 = W1_ref[...]
    xs1_ref[...] = jnp.dot(xf, W1[:H], preferred_element_type=F32)
    xs2_ref[...] = jnp.dot(xf, W1[H:2 * H], preferred_element_type=F32)


_unpool_call = pl.pallas_call(
    _unpool_body,
    grid=(N1 // BLK_N,),
    in_specs=[
        pl.BlockSpec((N0, H), lambda i: (0, 0)),
        pl.BlockSpec((1, N0), lambda i: (0, 0)),
        pl.BlockSpec((BLK_N, H), lambda i: (i, 0)),
        pl.BlockSpec((3 * H, H), lambda i: (0, 0)),
    ],
    out_specs=[pl.BlockSpec((BLK_N, H), lambda i: (i, 0))] * 3,
    out_shape=[jax.ShapeDtypeStruct((N1, H), F32)] * 3,
)


@functools.cache
def _make_sc_gather():
    mesh = plsc.VectorSubcoreMesh(
        core_axis_name="c", subcore_axis_name="s",
        num_cores=NC, num_subcores=NS)

    @functools.partial(
        pl.kernel,
        out_type=(jax.ShapeDtypeStruct((E1, H), F32),
                  jax.ShapeDtypeStruct((E1, H), F32)),
        mesh=mesh,
        scratch_types=[
            pltpu.VMEM((CH,), I32),
            pltpu.VMEM((CH,), I32),
            pltpu.VMEM((CH, H), F32),
            pltpu.VMEM((CH, H), F32),
            pltpu.SemaphoreType.DMA,
            pltpu.SemaphoreType.DMA,
        ],
    )
    def sc_gather(xs1_hbm, xs2_hbm, src_hbm, dst_hbm, g1_hbm, g2_hbm,
                  sidx, didx, buf1, buf2, sem1, sem2):
        wid = lax.axis_index("s") * NC + lax.axis_index("c")
        base = wid * EPW

        def body(j, carry):
            off = base + j * CH
            pltpu.sync_copy(src_hbm.at[pl.ds(off, CH)], sidx)
            pltpu.sync_copy(dst_hbm.at[pl.ds(off, CH)], didx)
            cp1 = pltpu.async_copy(xs1_hbm.at[sidx], buf1, sem1)
            cp2 = pltpu.async_copy(xs2_hbm.at[didx], buf2, sem2)
            cp1.wait()
            cp2.wait()
            pltpu.sync_copy(buf1, g1_hbm.at[pl.ds(off, CH)])
            pltpu.sync_copy(buf2, g2_hbm.at[pl.ds(off, CH)])
            return carry

        lax.fori_loop(0, NCHUNK, body, 0)

    return sc_gather


def _edge_body(g1_ref, g2_ref, ea_ref, W1_ref, b1_ref, W2_ref, b2_ref,
               g_ref, bn_ref, out_ref):
    ea = ea_ref[...]
    h = _elu(g1_ref[...] + g2_ref[...]
             + jnp.dot(ea, W1_ref[...][2 * H:], preferred_element_type=F32)
             + b1_ref[...])
    h = jnp.dot(h, W2_ref[...], preferred_element_type=F32) + b2_ref[...]
    out_ref[...] = _ln(ea + h, g_ref[...], bn_ref[...])


_edge_call = pl.pallas_call(
    _edge_body,
    grid=(E1 // BLK_E,),
    in_specs=[
        pl.BlockSpec((BLK_E, H), lambda i: (i, 0)),
        pl.BlockSpec((BLK_E, H), lambda i: (i, 0)),
        pl.BlockSpec((BLK_E, H), lambda i: (i, 0)),
        pl.BlockSpec((3 * H, H), lambda i: (0, 0)),
        pl.BlockSpec((1, H), lambda i: (0, 0)),
        pl.BlockSpec((H, H), lambda i: (0, 0)),
        pl.BlockSpec((1, H), lambda i: (0, 0)),
        pl.BlockSpec((1, H), lambda i: (0, 0)),
        pl.BlockSpec((1, H), lambda i: (0, 0)),
    ],
    out_specs=pl.BlockSpec((BLK_E, H), lambda i: (i, 0)),
    out_shape=jax.ShapeDtypeStruct((E1, H), F32),
)


EPW1 = E1 // NS
CHS = 80
NCHUNKS = EPW1 // CHS
RPT = (N1 // NS) // 8 * 8
CPR = 48
NTAIL = N1 - RPT * NS


@functools.cache
def _make_sc_scatter():
    mesh = plsc.VectorSubcoreMesh(
        core_axis_name="c", subcore_axis_name="s",
        num_cores=NC, num_subcores=NS)

    @functools.partial(
        pl.kernel,
        out_type=(jax.ShapeDtypeStruct((NC, N1, H), F32),
                  jax.ShapeDtypeStruct((NC, N1, H), F32)),
        mesh=mesh,
        scratch_types=[
            pltpu.VMEM((CHS,), I32),
            pltpu.VMEM((CHS, H), F32),
            pltpu.VMEM((CHS, H), F32),
            pltpu.VMEM((CPR,), I32),
            pltpu.VMEM((NTAIL,), I32),
            pltpu.VMEM((CPR, H), F32),
            pltpu.VMEM_SHARED((N1, H), F32),
        ],
    )
    def sc_scatter(ea1_hbm, dst_hbm, iota_hbm, zacc_hbm, ones_hbm,
                   acc_out, cnt_out, idxb, buf, onesb, ridx, ridx2, abuf,
                   acc_sh):
        c = lax.axis_index("c")
        s = lax.axis_index("s")
        r0 = s * RPT
        base = (c * NS + s) * EPW

        def zero_fill():
            for k in range(RPT // CPR):
                pltpu.sync_copy(iota_hbm.at[pl.ds(r0 + k * CPR, CPR)], ridx)
                pltpu.sync_copy(abuf, acc_sh.at[ridx])

            @pl.when(s == NS - 1)
            def _():
                t0 = RPT * NS
                pltpu.sync_copy(iota_hbm.at[pl.ds(t0, NTAIL)], ridx2)
                pltpu.sync_copy(abuf.at[pl.ds(0, NTAIL)], acc_sh.at[ridx2])

        def drain(out_hbm):
            for k in range(RPT // CPR):
                pltpu.sync_copy(iota_hbm.at[pl.ds(r0 + k * CPR, CPR)], ridx)
                pltpu.sync_copy(acc_sh.at[ridx], abuf)
                pltpu.sync_copy(abuf, out_hbm.at[c, pl.ds(r0 + k * CPR, CPR)])

            @pl.when(s == NS - 1)
            def _():
                t0 = RPT * NS
                pltpu.sync_copy(iota_hbm.at[pl.ds(t0, NTAIL)], ridx2)
                pltpu.sync_copy(acc_sh.at[ridx2], abuf.at[pl.ds(0, NTAIL)])
                pltpu.sync_copy(abuf.at[pl.ds(0, NTAIL)],
                                out_hbm.at[c, pl.ds(t0, NTAIL)])

        pltpu.sync_copy(zacc_hbm, abuf)
        zero_fill()
        plsc.subcore_barrier()

        def body(j, carry):
            off = base + j * CHS
            pltpu.sync_copy(dst_hbm.at[pl.ds(off, CHS)], idxb)
            pltpu.sync_copy(ea1_hbm.at[pl.ds(off, CHS)], buf)
            pltpu.sync_copy(buf, acc_sh.at[idxb], add=True)
            return carry

        lax.fori_loop(0, NCHUNK, body, 0)
        plsc.subcore_barrier()
        drain(acc_out)

        plsc.subcore_barrier()
        pltpu.sync_copy(zacc_hbm, abuf)
        zero_fill()
        pltpu.sync_copy(ones_hbm, onesb)
        plsc.subcore_barrier()

        def body2(j, carry):
            off = base + j * CHS
            pltpu.sync_copy(dst_hbm.at[pl.ds(off, CHS)], idxb)
            pltpu.sync_copy(onesb, acc_sh.at[idxb], add=True)
            return carry

        lax.fori_loop(0, NCHUNK, body2, 0)
        plsc.subcore_barrier()
        drain(cnt_out)

    return sc_scatter


def _decode_body(xf_ref, acc_ref, cnt_ref, W1_ref, b1_ref, W2_ref,
                 b2_ref, g_ref, bn_ref, dW1_ref, db1_ref, dW2_ref, db2_ref,
                 out_ref):
    xf = xf_ref[...]
    cnt = jnp.maximum(cnt_ref[0, :, 0:1] + cnt_ref[1, :, 0:1], 1.0)
    agg = (acc_ref[0] + acc_ref[1]) / cnt
    W1 = W1_ref[...]
    h = _elu(jnp.dot(xf, W1[:H], preferred_element_type=F32)
             + jnp.dot(agg, W1[H:], preferred_element_type=F32)
             + b1_ref[...])
    h = jnp.dot(h, W2_ref[...], preferred_element_type=F32) + b2_ref[...]
    xf2 = _ln(xf + h, g_ref[...], bn_ref[...])
    d = _elu(jnp.dot(xf2, dW1_ref[...], preferred_element_type=F32) + db1_ref[...])
    out_ref[...] = jnp.dot(d, dW2_ref[...], preferred_element_type=F32) + db2_ref[...]


_decode_call = pl.pallas_call(
    _decode_body,
    grid=(N1 // BLK_N,),
    in_specs=[
        pl.BlockSpec((BLK_N, H), lambda i: (i, 0)),
        pl.BlockSpec((NC, BLK_N, H), lambda i: (0, i, 0)),
        pl.BlockSpec((NC, BLK_N, H), lambda i: (0, i, 0)),
        pl.BlockSpec((2 * H, H), lambda i: (0, 0)),
        pl.BlockSpec((1, H), lambda i: (0, 0)),
        pl.BlockSpec((H, H), lambda i: (0, 0)),
        pl.BlockSpec((1, H), lambda i: (0, 0)),
        pl.BlockSpec((1, H), lambda i: (0, 0)),
        pl.BlockSpec((1, H), lambda i: (0, 0)),
        pl.BlockSpec((H, H), lambda i: (0, 0)),
        pl.BlockSpec((1, H), lambda i: (0, 0)),
        pl.BlockSpec((H, H), lambda i: (0, 0)),
        pl.BlockSpec((1, H), lambda i: (0, 0)),
    ],
    out_specs=pl.BlockSpec((BLK_N, H), lambda i: (i, 0)),
    out_shape=jax.ShapeDtypeStruct((N1, H), F32),
)


def _sc_gather_fn(xs1, xs2, src, dst):
    return _make_sc_gather()(xs1, xs2, src, dst)


def _sc_scatter_fn(ea1, dst, iota, zacc, ones):
    return _make_sc_scatter()(ea1, dst, iota, zacc, ones)


def kernel(x, edge_index, edge_attr, x_fine_skip, edge_index_fine,
           edge_attr_fine, perm, params):
    p = params
    src0 = edge_index[0].astype(I32)
    dst0 = edge_index[1].astype(I32)
    srcf = edge_index_fine[0].astype(I32)
    dstf = edge_index_fine[1].astype(I32)
    perm = perm.astype(I32)

    x_flat8 = jnp.concatenate(
        [x.reshape(1, N0 * H), jnp.zeros((7, N0 * H), F32)], axis=0)
    s_ae = _ae_call(x_flat8, p['ae_W1'])

    x0f = _coarse_call(
        x, src0.reshape(E0, 1), dst0.reshape(E0, 1), dst0.reshape(1, E0),
        edge_attr, s_ae, perm.reshape(N0, 1), perm.reshape(1, N0),
        p['e0_W1'], p['e0_b1'].reshape(1, H), p['e0_W2'],
        p['e0_b2'].reshape(1, H), p['e0_g'].reshape(1, H),
        p['e0_bn'].reshape(1, H),
        p['ae_b1'].reshape(1, AE1), p['ae_W2'], p['ae_b2'].reshape(1, ENC),
        p['attn_w'][:H].reshape(H, 1),
        p['n0_W1'], p['n0_b1'].reshape(1, H), p['n0_W2'],
        p['n0_b2'].reshape(1, H), p['n0_g'].reshape(1, H),
        p['n0_bn'].reshape(1, H))

    xf, xs1, xs2 = _unpool_call(x0f, perm.reshape(1, N0), x_fine_skip,
                                p['e1_W1'])

    g1, g2 = _sc_gather_fn(xs1, xs2, srcf, dstf)

    ea1 = _edge_call(g1, g2, edge_attr_fine, p['e1_W1'],
                     p['e1_b1'].reshape(1, H), p['e1_W2'],
                     p['e1_b2'].reshape(1, H), p['e1_g'].reshape(1, H),
                     p['e1_bn'].reshape(1, H))

    acc, cnt = _sc_scatter_fn(ea1, dstf, jnp.arange(N1, dtype=I32),
                              jnp.zeros((CPR, H), F32),
                              jnp.ones((CHS, H), F32))

    dW2p = jnp.zeros((H, H), F32).at[:, :OUT].set(p['dec_W2'])
    db2p = jnp.zeros((1, H), F32).at[0, :OUT].set(p['dec_b2'])
    out_full = _decode_call(
        xf, acc, cnt,
        p['n1_W1'], p['n1_b1'].reshape(1, H), p['n1_W2'],
        p['n1_b2'].reshape(1, H), p['n1_g'].reshape(1, H),
        p['n1_bn'].reshape(1, H), p['dec_W1'], p['dec_b1'].reshape(1, H),
        dW2p, db2p)
    return out_full[:, :OUT]


# speedup vs baseline: 2.9153x; 1.0012x over previous
import functools

import jax
import jax.numpy as jnp
from jax import lax
from jax.experimental import pallas as pl
from jax.experimental.pallas import tpu as pltpu
from jax.experimental.pallas import tpu_sc as plsc

F32 = jnp.float32
I32 = jnp.int32

N0, N1 = 256, 10000
E0, E1 = 8192, 320000
H = 128
ENC = 64
OUT = 3
AE1 = 256
EPS = 1e-5

NC, NS = 2, 16
NW = NC * NS
EPW = E1 // NW
CH = 80
NCHUNK = EPW // CH

BLK_E = 3200
BLK_N = 1000


def _elu(v):
    return jnp.where(v > 0, v, jnp.exp(v) - 1.0)


def _ln(t, g, b):
    mu = jnp.mean(t, axis=-1, keepdims=True)
    var = jnp.mean((t - mu) ** 2, axis=-1, keepdims=True)
    return (t - mu) / jnp.sqrt(var + EPS) * g + b


def _ae_body(xf_ref, w_ref, acc_ref):
    @pl.when(pl.program_id(0) == 0)
    def _():
        acc_ref[...] = jnp.zeros_like(acc_ref)

    acc_ref[...] += jnp.dot(xf_ref[...], w_ref[...], preferred_element_type=F32)


_ae_call = pl.pallas_call(
    _ae_body,
    grid=(16,),
    in_specs=[
        pl.BlockSpec((8, 2048), lambda k: (0, k)),
        pl.BlockSpec((2048, AE1), lambda k: (k, 0)),
    ],
    out_specs=pl.BlockSpec((8, AE1), lambda k: (0, 0)),
    out_shape=jax.ShapeDtypeStruct((8, AE1), F32),
)


def _coarse_body(x_ref, srcc_ref, dstc_ref, dstr_ref, ea_ref, s_ref,
                 permc_ref, permr_ref,
                 e0W1_ref, e0b1_ref, e0W2_ref, e0b2_ref, e0g_ref, e0bn_ref,
                 aeb1_ref, aeW2_ref, aeb2_ref, attnx_ref,
                 n0W1_ref, n0b1_ref, n0W2_ref, n0b2_ref, n0g_ref, n0bn_ref,
                 out_ref):
    x = x_ref[...]
    W1 = e0W1_ref[...]
    xa = jnp.dot(x, W1[:H], preferred_element_type=F32)
    xb = jnp.dot(x, W1[H:2 * H], preferred_element_type=F32)
    col = lax.broadcasted_iota(I32, (E0, N0), 1)
    oh_src = (srcc_ref[...] == col).astype(F32)
    oh_dst = (dstc_ref[...] == col).astype(F32)
    row = lax.broadcasted_iota(I32, (N0, E0), 0)
    oh_dstT = (dstr_ref[...] == row).astype(F32)
    ea = ea_ref[...]
    h = _elu(jnp.dot(oh_src, xa, preferred_element_type=F32)
             + jnp.dot(oh_dst, xb, preferred_element_type=F32)
             + jnp.dot(ea, W1[2 * H:], preferred_element_type=F32)
             + e0b1_ref[...])
    h = jnp.dot(h, e0W2_ref[...], preferred_element_type=F32) + e0b2_ref[...]
    ea0 = _ln(ea + h, e0g_ref[...], e0bn_ref[...])
    sums = jnp.dot(oh_dstT, ea0, preferred_element_type=F32)
    cnt = jnp.maximum(jnp.sum(oh_dstT, axis=-1, keepdims=True), 1.0)
    agg0 = sums / cnt
    x_ae = _elu(s_ref[0:1, :] + aeb1_ref[...])
    g_ae = jnp.dot(x_ae, aeW2_ref[...], preferred_element_type=F32) + aeb2_ref[...]
    logit = jnp.dot(x, attnx_ref[...], preferred_element_type=F32)
    e = jnp.exp(logit - jnp.max(logit))
    score = e / jnp.sum(e)
    n0W1 = n0W1_ref[...]
    pre = (jnp.dot(x, n0W1[:H], preferred_element_type=F32)
           + jnp.dot(agg0, n0W1[H:2 * H], preferred_element_type=F32)
           + score * jnp.dot(g_ae, n0W1[2 * H:], preferred_element_type=F32)
           + n0b1_ref[...])
    h = _elu(pre)
    h = jnp.dot(h, n0W2_ref[...], preferred_element_type=F32) + n0b2_ref[...]
    x0 = _ln(x + h, n0g_ref[...], n0bn_ref[...])
    pc = permc_ref[...]
    pr = permr_ref[...]
    jj = lax.broadcasted_iota(I32, (N0, N0), 1)
    winidx = jnp.max(jnp.where(pc == pr, jj, -1), axis=-1, keepdims=True)
    win = (lax.broadcasted_iota(I32, (N0, 1), 0) == winidx).astype(F32)
    out_ref[...] = x0 * win


_coarse_shapes = [
    (N0, H), (E0, 1), (E0, 1), (1, E0), (E0, H), (8, AE1),
    (N0, 1), (1, N0),
    (3 * H, H), (1, H), (H, H), (1, H), (1, H), (1, H),
    (1, AE1), (AE1, ENC), (1, ENC), (H, 1),
    (2 * H + ENC, H), (1, H), (H, H), (1, H), (1, H), (1, H),
]

_coarse_call = pl.pallas_call(
    _coarse_body,
    in_specs=[pl.BlockSpec(s, lambda: (0,) * len(s)) for s in _coarse_shapes],
    out_specs=pl.BlockSpec((N0, H), lambda: (0, 0)),
    out_shape=jax.ShapeDtypeStruct((N0, H), F32),
)


def _unpool_body(x0_ref, permr_ref, skip_ref, W1_ref, xf_ref, xs1_ref, xs2_ref):
    b = pl.program_id(0)
    rows = lax.broadcasted_iota(I32, (BLK_N, N0), 0) + b * BLK_N
    oh = (rows == permr_ref[...]).astype(F32)
    xf = jnp.dot(oh, x0_ref[...], preferred_element_type=F32) + skip_ref[...]
    xf_ref[...] = xf
    W1 = W1_ref[...]
    xs1_ref[...] = jnp.dot(xf, W1[:H], preferred_element_type=F32)
    xs2_ref[...] = jnp.dot(xf, W1[H:2 * H], preferred_element_type=F32)


_unpool_call = pl.pallas_call(
    _unpool_body,
    grid=(N1 // BLK_N,),
    in_specs=[
        pl.BlockSpec((N0, H), lambda i: (0, 0)),
        pl.BlockSpec((1, N0), lambda i: (0, 0)),
        pl.BlockSpec((BLK_N, H), lambda i: (i, 0)),
        pl.BlockSpec((3 * H, H), lambda i: (0, 0)),
    ],
    out_specs=[pl.BlockSpec((BLK_N, H), lambda i: (i, 0))] * 3,
    out_shape=[jax.ShapeDtypeStruct((N1, H), F32)] * 3,
)


@functools.cache
def _make_sc_gather():
    mesh = plsc.VectorSubcoreMesh(
        core_axis_name="c", subcore_axis_name="s",
        num_cores=NC, num_subcores=NS)

    @functools.partial(
        pl.kernel,
        out_type=(jax.ShapeDtypeStruct((E1, H), F32),
                  jax.ShapeDtypeStruct((E1, H), F32)),
        mesh=mesh,
        scratch_types=[
            pltpu.VMEM((CH,), I32),
            pltpu.VMEM((CH,), I32),
            pltpu.VMEM((CH, H), F32),
            pltpu.VMEM((CH, H), F32),
            pltpu.SemaphoreType.DMA,
            pltpu.SemaphoreType.DMA,
        ],
    )
    def sc_gather(xs1_hbm, xs2_hbm, src_hbm, dst_hbm, g1_hbm, g2_hbm,
                  sidx, didx, buf1, buf2, sem1, sem2):
        wid = lax.axis_index("s") * NC + lax.axis_index("c")
        base = wid * EPW

        def body(j, carry):
            off = base + j * CH
            pltpu.sync_copy(src_hbm.at[pl.ds(off, CH)], sidx)
            pltpu.sync_copy(dst_hbm.at[pl.ds(off, CH)], didx)
            cp1 = pltpu.async_copy(xs1_hbm.at[sidx], buf1, sem1)
            cp2 = pltpu.async_copy(xs2_hbm.at[didx], buf2, sem2)
            cp1.wait()
            cp2.wait()
            pltpu.sync_copy(buf1, g1_hbm.at[pl.ds(off, CH)])
            pltpu.sync_copy(buf2, g2_hbm.at[pl.ds(off, CH)])
            return carry

        lax.fori_loop(0, NCHUNK, body, 0)

    return sc_gather


def _edge_body(g1_ref, g2_ref, ea_ref, W1_ref, b1_ref, W2_ref, b2_ref,
               g_ref, bn_ref, out_ref):
    ea = ea_ref[...]
    h = _elu(g1_ref[...] + g2_ref[...]
             + jnp.dot(ea, W1_ref[...][2 * H:], preferred_element_type=F32)
             + b1_ref[...])
    h = jnp.dot(h, W2_ref[...], preferred_element_type=F32) + b2_ref[...]
    out_ref[...] = _ln(ea + h, g_ref[...], bn_ref[...])


_edge_call = pl.pallas_call(
    _edge_body,
    grid=(E1 // BLK_E,),
    in_specs=[
        pl.BlockSpec((BLK_E, H), lambda i: (i, 0)),
        pl.BlockSpec((BLK_E, H), lambda i: (i, 0)),
        pl.BlockSpec((BLK_E, H), lambda i: (i, 0)),
        pl.BlockSpec((3 * H, H), lambda i: (0, 0)),
        pl.BlockSpec((1, H), lambda i: (0, 0)),
        pl.BlockSpec((H, H), lambda i: (0, 0)),
        pl.BlockSpec((1, H), lambda i: (0, 0)),
        pl.BlockSpec((1, H), lambda i: (0, 0)),
        pl.BlockSpec((1, H), lambda i: (0, 0)),
    ],
    out_specs=pl.BlockSpec((BLK_E, H), lambda i: (i, 0)),
    out_shape=jax.ShapeDtypeStruct((E1, H), F32),
)


CHS = 80
RPT = (N1 // NS) // 8 * 8
CPR = 48
NTAIL = N1 - RPT * NS


@functools.cache
def _make_sc_scatter():
    mesh = plsc.VectorSubcoreMesh(
        core_axis_name="c", subcore_axis_name="s",
        num_cores=NC, num_subcores=NS)

    @functools.partial(
        pl.kernel,
        out_type=(jax.ShapeDtypeStruct((NC, N1, H), F32),
                  jax.ShapeDtypeStruct((NC, N1, H), F32)),
        mesh=mesh,
        scratch_types=[
            pltpu.VMEM((CHS,), I32),
            pltpu.VMEM((CHS, H), F32),
            pltpu.VMEM((CHS, H), F32),
            pltpu.VMEM((CPR,), I32),
            pltpu.VMEM((NTAIL,), I32),
            pltpu.VMEM((CPR, H), F32),
            pltpu.VMEM_SHARED((N1, H), F32),
        ],
    )
    def sc_scatter(ea1_hbm, dst_hbm, iota_hbm, zacc_hbm, ones_hbm,
                   acc_out, cnt_out, idxb, buf, onesb, ridx, ridx2, abuf,
                   acc_sh):
        c = lax.axis_index("c")
        s = lax.axis_index("s")
        r0 = s * RPT
        base = (c * NS + s) * EPW

        def zero_fill():
            for k in range(RPT // CPR):
                pltpu.sync_copy(iota_hbm.at[pl.ds(r0 + k * CPR, CPR)], ridx)
                pltpu.sync_copy(abuf, acc_sh.at[ridx])

            @pl.when(s == NS - 1)
            def _():
                t0 = RPT * NS
                pltpu.sync_copy(iota_hbm.at[pl.ds(t0, NTAIL)], ridx2)
                pltpu.sync_copy(abuf.at[pl.ds(0, NTAIL)], acc_sh.at[ridx2])

        def drain(out_hbm):
            for k in range(RPT // CPR):
                pltpu.sync_copy(iota_hbm.at[pl.ds(r0 + k * CPR, CPR)], ridx)
                pltpu.sync_copy(acc_sh.at[ridx], abuf)
                pltpu.sync_copy(abuf, out_hbm.at[c, pl.ds(r0 + k * CPR, CPR)])

            @pl.when(s == NS - 1)
            def _():
                t0 = RPT * NS
                pltpu.sync_copy(iota_hbm.at[pl.ds(t0, NTAIL)], ridx2)
                pltpu.sync_copy(acc_sh.at[ridx2], abuf.at[pl.ds(0, NTAIL)])
                pltpu.sync_copy(abuf.at[pl.ds(0, NTAIL)],
                                out_hbm.at[c, pl.ds(t0, NTAIL)])

        pltpu.sync_copy(zacc_hbm, abuf)
        zero_fill()
        plsc.subcore_barrier()

        def body(j, carry):
            off = base + j * CHS
            pltpu.sync_copy(dst_hbm.at[pl.ds(off, CHS)], idxb)
            pltpu.sync_copy(ea1_hbm.at[pl.ds(off, CHS)], buf)
            pltpu.sync_copy(buf, acc_sh.at[idxb], add=True)
            return carry

        lax.fori_loop(0, NCHUNK, body, 0)
        plsc.subcore_barrier()
        drain(acc_out)

        plsc.subcore_barrier()
        pltpu.sync_copy(zacc_hbm, abuf)
        zero_fill()
        pltpu.sync_copy(ones_hbm, onesb)
        plsc.subcore_barrier()

        def body2(j, carry):
            off = base + j * CHS
            pltpu.sync_copy(dst_hbm.at[pl.ds(off, CHS)], idxb)
            pltpu.sync_copy(onesb, acc_sh.at[idxb], add=True)
            return carry

        lax.fori_loop(0, NCHUNK, body2, 0)
        plsc.subcore_barrier()
        drain(cnt_out)

    return sc_scatter


def _decode_body(xf_ref, acc_ref, cnt_ref, W1_ref, b1_ref, W2_ref,
                 b2_ref, g_ref, bn_ref, dW1_ref, db1_ref, dW2_ref, db2_ref,
                 out_ref):
    xf = xf_ref[...]
    cnt = jnp.maximum(cnt_ref[0, :, 0:1] + cnt_ref[1, :, 0:1], 1.0)
    agg = (acc_ref[0] + acc_ref[1]) / cnt
    W1 = W1_ref[...]
    h = _elu(jnp.dot(xf, W1[:H], preferred_element_type=F32)
             + jnp.dot(agg, W1[H:], preferred_element_type=F32)
             + b1_ref[...])
    h = jnp.dot(h, W2_ref[...], preferred_element_type=F32) + b2_ref[...]
    xf2 = _ln(xf + h, g_ref[...], bn_ref[...])
    d = _elu(jnp.dot(xf2, dW1_ref[...], preferred_element_type=F32) + db1_ref[...])
    out_ref[...] = jnp.dot(d, dW2_ref[...], preferred_element_type=F32) + db2_ref[...]


_decode_call = pl.pallas_call(
    _decode_body,
    grid=(N1 // BLK_N,),
    in_specs=[
        pl.BlockSpec((BLK_N, H), lambda i: (i, 0)),
        pl.BlockSpec((NC, BLK_N, H), lambda i: (0, i, 0)),
        pl.BlockSpec((NC, BLK_N, H), lambda i: (0, i, 0)),
        pl.BlockSpec((2 * H, H), lambda i: (0, 0)),
        pl.BlockSpec((1, H), lambda i: (0, 0)),
        pl.BlockSpec((H, H), lambda i: (0, 0)),
        pl.BlockSpec((1, H), lambda i: (0, 0)),
        pl.BlockSpec((1, H), lambda i: (0, 0)),
        pl.BlockSpec((1, H), lambda i: (0, 0)),
        pl.BlockSpec((H, H), lambda i: (0, 0)),
        pl.BlockSpec((1, H), lambda i: (0, 0)),
        pl.BlockSpec((H, H), lambda i: (0, 0)),
        pl.BlockSpec((1, H), lambda i: (0, 0)),
    ],
    out_specs=pl.BlockSpec((BLK_N, H), lambda i: (i, 0)),
    out_shape=jax.ShapeDtypeStruct((N1, H), F32),
)


def _sc_gather_fn(xs1, xs2, src, dst):
    return _make_sc_gather()(xs1, xs2, src, dst)


def _sc_scatter_fn(ea1, dst, iota, zacc, ones):
    return _make_sc_scatter()(ea1, dst, iota, zacc, ones)


def kernel(x, edge_index, edge_attr, x_fine_skip, edge_index_fine,
           edge_attr_fine, perm, params):
    p = params
    src0 = edge_index[0].astype(I32)
    dst0 = edge_index[1].astype(I32)
    srcf = edge_index_fine[0].astype(I32)
    dstf = edge_index_fine[1].astype(I32)
    perm = perm.astype(I32)

    x_flat8 = jnp.concatenate(
        [x.reshape(1, N0 * H), jnp.zeros((7, N0 * H), F32)], axis=0)
    s_ae = _ae_call(x_flat8, p['ae_W1'])

    x0f = _coarse_call(
        x, src0.reshape(E0, 1), dst0.reshape(E0, 1), dst0.reshape(1, E0),
        edge_attr, s_ae, perm.reshape(N0, 1), perm.reshape(1, N0),
        p['e0_W1'], p['e0_b1'].reshape(1, H), p['e0_W2'],
        p['e0_b2'].reshape(1, H), p['e0_g'].reshape(1, H),
        p['e0_bn'].reshape(1, H),
        p['ae_b1'].reshape(1, AE1), p['ae_W2'], p['ae_b2'].reshape(1, ENC),
        p['attn_w'][:H].reshape(H, 1),
        p['n0_W1'], p['n0_b1'].reshape(1, H), p['n0_W2'],
        p['n0_b2'].reshape(1, H), p['n0_g'].reshape(1, H),
        p['n0_bn'].reshape(1, H))

    xf, xs1, xs2 = _unpool_call(x0f, perm.reshape(1, N0), x_fine_skip,
                                p['e1_W1'])

    g1, g2 = _sc_gather_fn(xs1, xs2, srcf, dstf)

    ea1 = _edge_call(g1, g2, edge_attr_fine, p['e1_W1'],
                     p['e1_b1'].reshape(1, H), p['e1_W2'],
                     p['e1_b2'].reshape(1, H), p['e1_g'].reshape(1, H),
                     p['e1_bn'].reshape(1, H))

    acc, cnt = _sc_scatter_fn(ea1, dstf, jnp.arange(N1, dtype=I32),
                              jnp.zeros((CPR, H), F32),
                              jnp.ones((CHS, H), F32))

    dW2p = jnp.zeros((H, H), F32).at[:, :OUT].set(p['dec_W2'])
    db2p = jnp.zeros((1, H), F32).at[0, :OUT].set(p['dec_b2'])
    out_full = _decode_call(
        xf, acc, cnt,
        p['n1_W1'], p['n1_b1'].reshape(1, H), p['n1_W2'],
        p['n1_b2'].reshape(1, H), p['n1_g'].reshape(1, H),
        p['n1_bn'].reshape(1, H), p['dec_W1'], p['dec_b1'].reshape(1, H),
        dW2p, db2p)
    return out_full[:, :OUT]
